# Initial kernel scaffold; baseline (speedup 1.0000x reference)
import jax
import jax.numpy as jnp
from jax.experimental import pallas as pl


def kernel(x, edge_index, W1, b1, W2, b2):
    raise NotImplementedError("write your pallas kernel here")


# speedup vs baseline: 9.7808x; 9.7808x over previous
import functools

import jax
import jax.numpy as jnp
from jax import lax
from jax.experimental import pallas as pl
from jax.experimental.pallas import tpu as pltpu
from jax.experimental.pallas import tpu_sc as plsc

N = 10000
E = 160000
D = 256
H = 128
OUT = 2

NC = 2
NS = 16
NW = NC * NS
CHUNK = 128
EW = E // NW
FULL_CHUNKS = EW // CHUNK
TAIL = EW - FULL_CHUNKS * CHUNK
NPAD = 10240
RPT = NPAD // NS

_mesh = lambda: plsc.VectorSubcoreMesh(core_axis_name="c", subcore_axis_name="s")


DW = 128


def _make_deg_kernel():
    @functools.partial(
        pl.kernel,
        out_type=jax.ShapeDtypeStruct((NC, NPAD, DW), jnp.float32),
        mesh=_mesh(),
        scratch_types=[
            pltpu.VMEM((CHUNK,), jnp.int32),
            pltpu.VMEM((CHUNK, DW), jnp.float32),
            pltpu.VMEM((TAIL,), jnp.int32),
            pltpu.VMEM((TAIL, DW), jnp.float32),
            pltpu.VMEM_SHARED((NPAD, DW), jnp.float32),
        ],
    )
    def deg_kernel(dst_hbm, ones_hbm, z1d_hbm, out_hbm, didx, ones, didx_t, ones_t, acc):
        c = lax.axis_index("c")
        s = lax.axis_index("s")
        wid = s * NC + c
        r0 = pl.multiple_of(s * RPT, 8)
        pltpu.sync_copy(z1d_hbm, acc.at[pl.ds(r0, RPT)])
        pltpu.sync_copy(ones_hbm, ones)
        pltpu.sync_copy(ones_hbm.at[pl.ds(0, TAIL)], ones_t)
        plsc.subcore_barrier()

        base0 = wid * EW

        def body(i, _):
            base = pl.multiple_of(base0 + i * CHUNK, 8)
            pltpu.sync_copy(dst_hbm.at[pl.ds(base, CHUNK)], didx)
            pltpu.sync_copy(ones, acc.at[didx], add=True)
            return 0

        lax.fori_loop(0, FULL_CHUNKS, body, 0)
        if TAIL:
            base = pl.multiple_of(base0 + FULL_CHUNKS * CHUNK, 8)
            pltpu.sync_copy(dst_hbm.at[pl.ds(base, TAIL)], didx_t)
            pltpu.sync_copy(ones_t, acc.at[didx_t], add=True)
        plsc.subcore_barrier()
        pltpu.sync_copy(acc.at[pl.ds(r0, RPT)], out_hbm.at[c, pl.ds(r0, RPT)])

    return deg_kernel


def _make_agg_kernel(F):
    @functools.partial(
        pl.kernel,
        out_type=jax.ShapeDtypeStruct((NC, NPAD, F), jnp.float32),
        mesh=_mesh(),
        scratch_types=[
            pltpu.VMEM((CHUNK,), jnp.int32),
            pltpu.VMEM((CHUNK,), jnp.int32),
            pltpu.VMEM((CHUNK, F), jnp.float32),
            pltpu.VMEM((TAIL,), jnp.int32),
            pltpu.VMEM((TAIL,), jnp.int32),
            pltpu.VMEM((TAIL, F), jnp.float32),
            pltpu.VMEM_SHARED((NPAD, F), jnp.float32),
            pltpu.SemaphoreType.DMA,
        ],
    )
    def agg_kernel(tbl_hbm, src_hbm, dst_hbm, z2d_hbm, out_hbm,
                   sidx, didx, rows, sidx_t, didx_t, rows_t, acc, gsem):
        c = lax.axis_index("c")
        s = lax.axis_index("s")
        wid = s * NC + c
        r0 = pl.multiple_of(s * RPT, 8)
        pltpu.sync_copy(z2d_hbm, acc.at[pl.ds(r0, RPT)])
        plsc.subcore_barrier()

        base0 = wid * EW

        def body(i, _):
            base = pl.multiple_of(base0 + i * CHUNK, 8)
            pltpu.sync_copy(src_hbm.at[pl.ds(base, CHUNK)], sidx)
            pltpu.sync_copy(dst_hbm.at[pl.ds(base, CHUNK)], didx)
            pltpu.async_copy(tbl_hbm.at[sidx], rows, gsem).wait()
            pltpu.sync_copy(rows, acc.at[didx], add=True)
            return 0

        lax.fori_loop(0, FULL_CHUNKS, body, 0)
        if TAIL:
            base = pl.multiple_of(base0 + FULL_CHUNKS * CHUNK, 8)
            pltpu.sync_copy(src_hbm.at[pl.ds(base, TAIL)], sidx_t)
            pltpu.sync_copy(dst_hbm.at[pl.ds(base, TAIL)], didx_t)
            pltpu.async_copy(tbl_hbm.at[sidx_t], rows_t, gsem).wait()
            pltpu.sync_copy(rows_t, acc.at[didx_t], add=True)
        plsc.subcore_barrier()
        pltpu.sync_copy(acc.at[pl.ds(r0, RPT)], out_hbm.at[c, pl.ds(r0, RPT)])

    return agg_kernel


_BR = 1000


def _dinv_from_degT(degT_blk):
    d = degT_blk[:, 0:1] + degT_blk[:, 1:2] + 1.0
    y = lax.rsqrt(d)
    return y * (1.5 - 0.5 * d * y * y)


def _tc1_body(x_ref, w1_ref, degT_ref, hs1_ref):
    dinv = _dinv_from_degT(degT_ref[...])
    h = jnp.dot(x_ref[...], w1_ref[...], preferred_element_type=jnp.float32,
                 precision=lax.Precision.HIGHEST)
    hs1_ref[...] = h * dinv


def _tc2_body(acc_ref, hs1_ref, degT_ref, b1_ref, g_ref):
    dinv = _dinv_from_degT(degT_ref[...])
    pre = (acc_ref[0] + acc_ref[1] + hs1_ref[...]) * dinv + b1_ref[...]
    z = jnp.maximum(pre, 0.0)
    g_ref[...] = z * dinv


def _tc3_body(acc_ref, g_ref, degT_ref, w2_ref, b2_ref, out_ref):
    dinv = _dinv_from_degT(degT_ref[...])
    tot = acc_ref[0] + acc_ref[1] + g_ref[...]
    h2 = jnp.dot(tot, w2_ref[...], preferred_element_type=jnp.float32,
                 precision=lax.Precision.HIGHEST)
    out_ref[...] = h2 * dinv + b2_ref[...]


def kernel(x, edge_index, W1, b1, W2, b2):
    x = x.astype(jnp.float32)
    src = edge_index[0]
    dst = edge_index[1]
    ones2d = jnp.ones((CHUNK, DW), jnp.float32)
    z2d_h = jnp.zeros((RPT, H), jnp.float32)
    z1d = z2d_h

    deg_p = _make_deg_kernel()(dst, ones2d, z1d)
    degT = deg_p[:, :N, 0].T

    hs1 = pl.pallas_call(
        _tc1_body,
        grid=(N // _BR,),
        in_specs=[
            pl.BlockSpec((_BR, D), lambda i: (i, 0)),
            pl.BlockSpec((D, H), lambda i: (0, 0)),
            pl.BlockSpec((_BR, 2), lambda i: (i, 0)),
        ],
        out_specs=pl.BlockSpec((_BR, H), lambda i: (i, 0)),
        out_shape=jax.ShapeDtypeStruct((N, H), jnp.float32),
    )(x, W1, degT)

    acc1 = _make_agg_kernel(H)(hs1, src, dst, z2d_h)[:, :N]

    g = pl.pallas_call(
        _tc2_body,
        grid=(N // _BR,),
        in_specs=[
            pl.BlockSpec((NC, _BR, H), lambda i: (0, i, 0)),
            pl.BlockSpec((_BR, H), lambda i: (i, 0)),
            pl.BlockSpec((_BR, 2), lambda i: (i, 0)),
            pl.BlockSpec((1, H), lambda i: (0, 0)),
        ],
        out_specs=pl.BlockSpec((_BR, H), lambda i: (i, 0)),
        out_shape=jax.ShapeDtypeStruct((N, H), jnp.float32),
    )(acc1, hs1, degT, b1.reshape(1, H))

    acc2 = _make_agg_kernel(H)(g, src, dst, z2d_h)[:, :N]

    out = pl.pallas_call(
        _tc3_body,
        grid=(N // _BR,),
        in_specs=[
            pl.BlockSpec((NC, _BR, H), lambda i: (0, i, 0)),
            pl.BlockSpec((_BR, H), lambda i: (i, 0)),
            pl.BlockSpec((_BR, 2), lambda i: (i, 0)),
            pl.BlockSpec((H, OUT), lambda i: (0, 0)),
            pl.BlockSpec((1, OUT), lambda i: (0, 0)),
        ],
        out_specs=pl.BlockSpec((_BR, OUT), lambda i: (i, 0)),
        out_shape=jax.ShapeDtypeStruct((N, OUT), jnp.float32),
    )(acc2, g, degT, W2, b2.reshape(1, OUT))

    return out
